# Initial kernel scaffold; baseline (speedup 1.0000x reference)
import jax
import jax.numpy as jnp
from jax.experimental import pallas as pl


def kernel(x, edge_index, edge_weight, W1, b1, Wg, Wih, bih, Whh, bhh, W2, b2):
    raise NotImplementedError("write your pallas kernel here")


# speedup vs baseline: 8.3256x; 8.3256x over previous
import functools

import jax
import jax.numpy as jnp
from jax import lax
from jax.experimental import pallas as pl
from jax.experimental.pallas import tpu as pltpu
from jax.experimental.pallas import tpu_sc as plsc

N = 10000
E = 320000
D_IN = 128
H = 64
C = 7

NC = 2
NS = 16
NW = NC * NS
CH = 128
NCHUNK = 80
EPW = NCHUNK * CH
E_PAD = NW * EPW
ROWS = E_PAD // CH
RPW = NCHUNK
RPT_ALL = ROWS // NS
N_PAD = 10240
NPT = N_PAD // NS
NROW_PT = N_PAD // NS

_mesh = plsc.VectorSubcoreMesh(core_axis_name="c", subcore_axis_name="s")
f32 = jnp.float32
i32 = jnp.int32


def _rsqrt_vec(x):
    i = plsc.bitcast(x, i32)
    i = 0x5F3759DF - (i >> 1)
    y = plsc.bitcast(i, f32)
    for _ in range(4):
        y = y * (1.5 - 0.5 * x * y * y)
    return y


@functools.partial(
    pl.kernel,
    compiler_params=pltpu.CompilerParams(
        needs_layout_passes=False, use_tc_tiling_on_sc=False),
    out_type=(
        jax.ShapeDtypeStruct((NC, N_PAD), f32),
        jax.ShapeDtypeStruct((ROWS, CH), f32),
    ),
    mesh=_mesh,
    scratch_types=[
        pltpu.VMEM((RPT_ALL, CH), i32),
        pltpu.VMEM((RPT_ALL, CH), f32),
        pltpu.VMEM((N_PAD,), f32),
        pltpu.VMEM((NPT,), f32),
        pltpu.VMEM((NPT,), f32),
        pltpu.VMEM((N_PAD,), f32),
        pltpu.VMEM((RPW, CH), i32),
        pltpu.VMEM((RPW, CH), i32),
        pltpu.VMEM((RPW, CH), f32),
        pltpu.VMEM((RPW, CH), f32),
        pltpu.VMEM_SHARED((NS, N_PAD), f32),
        pltpu.VMEM_SHARED((N_PAD,), f32),
    ],
)
def _sc_prep(src2d, dst2d, ew2d, dis_out, norm_out,
             dst_all, ew_all, deg_loc, red_buf, tmp_buf, dis_loc,
             src_w, dst_w, ew_w, norm_w, deg_sh, dis_sh):
    cid = lax.axis_index("c")
    sid = lax.axis_index("s")
    wid = cid * NS + sid

    r0 = sid * RPT_ALL
    pltpu.sync_copy(dst2d.at[pl.ds(r0, RPT_ALL)], dst_all)
    pltpu.sync_copy(ew2d.at[pl.ds(r0, RPT_ALL)], ew_all)

    def _zero_deg(v, _):
        deg_loc[pl.ds(v * 16, 16)] = jnp.zeros((16,), f32)
        return _
    lax.fori_loop(0, N_PAD // 16, _zero_deg, 0)

    def _deg_row(r, _):
        for j in range(CH // 16):
            d = dst_all[r, pl.ds(j * 16, 16)]
            w = ew_all[r, pl.ds(j * 16, 16)]
            plsc.addupdate_scatter(deg_loc, [d], w)
        return _
    lax.fori_loop(0, RPT_ALL, _deg_row, 0)

    pltpu.sync_copy(deg_loc, deg_sh.at[sid])
    plsc.subcore_barrier()

    off = sid * NPT
    pltpu.sync_copy(deg_sh.at[0, pl.ds(off, NPT)], red_buf)
    for t in range(1, NS):
        pltpu.sync_copy(deg_sh.at[t, pl.ds(off, NPT)], tmp_buf)
        def _acc(v, _, t=t):
            s = pl.ds(v * 16, 16)
            red_buf[s] = red_buf[s] + tmp_buf[s]
            return _
        lax.fori_loop(0, NPT // 16, _acc, 0)

    def _dis(v, _):
        s = pl.ds(v * 16, 16)
        red_buf[s] = _rsqrt_vec(red_buf[s] + 1.0)
        return _
    lax.fori_loop(0, NPT // 16, _dis, 0)

    pltpu.sync_copy(red_buf, dis_sh.at[pl.ds(off, NPT)])
    pltpu.sync_copy(red_buf, dis_out.at[cid, pl.ds(off, NPT)])
    plsc.subcore_barrier()
    pltpu.sync_copy(dis_sh, dis_loc)

    w0 = wid * RPW
    pltpu.sync_copy(src2d.at[pl.ds(w0, RPW)], src_w)
    pltpu.sync_copy(dst2d.at[pl.ds(w0, RPW)], dst_w)
    pltpu.sync_copy(ew2d.at[pl.ds(w0, RPW)], ew_w)

    def _norm_row(r, _):
        for j in range(CH // 16):
            s = pl.ds(j * 16, 16)
            sv = src_w[r, s]
            dv = dst_w[r, s]
            wv = ew_w[r, s]
            norm_w[r, s] = plsc.load_gather(dis_loc, [sv]) * wv * \
                plsc.load_gather(dis_loc, [dv])
        return _
    lax.fori_loop(0, RPW, _norm_row, 0)
    pltpu.sync_copy(norm_w, norm_out.at[pl.ds(w0, RPW)])


def _make_pass(width):
    nzc = NROW_PT // 128

    @functools.partial(
        pl.kernel,
        compiler_params=pltpu.CompilerParams(
            needs_layout_passes=False, use_tc_tiling_on_sc=False),
        out_type=jax.ShapeDtypeStruct((NC, N_PAD, width), f32),
        mesh=_mesh,
        scratch_types=[
            pltpu.VMEM((RPW, CH), i32),
            pltpu.VMEM((RPW, CH), i32),
            pltpu.VMEM((RPW, CH), f32),
            pltpu.VMEM((CH, width), f32),
            pltpu.VMEM((128, width), f32),
            pltpu.VMEM_SHARED((N_PAD, width), f32),
            pltpu.SemaphoreType.DMA,
        ],
    )
    def _pass(table, src2d, dst2d, coeff2d, out,
              src_w, dst_w, coeff_w, rows_v, zero_v, acc_sh, sem):
        cid = lax.axis_index("c")
        sid = lax.axis_index("s")
        wid = cid * NS + sid

        def _zrow(r, _):
            for j in range(width // 16):
                zero_v[r, pl.ds(j * 16, 16)] = jnp.zeros((16,), f32)
            return _
        lax.fori_loop(0, 128, _zrow, 0)
        for k in range(nzc):
            pltpu.sync_copy(zero_v, acc_sh.at[pl.ds(sid * NROW_PT + k * 128, 128)])
        plsc.subcore_barrier()

        w0 = wid * RPW
        pltpu.sync_copy(src2d.at[pl.ds(w0, RPW)], src_w)
        pltpu.sync_copy(dst2d.at[pl.ds(w0, RPW)], dst_w)
        pltpu.sync_copy(coeff2d.at[pl.ds(w0, RPW)], coeff_w)

        def _chunk(c, _):
            pltpu.async_copy(table.at[src_w.at[c]], rows_v, sem).wait()

            def _edge(k, _):
                cf = plsc.load_gather(
                    coeff_w, [jnp.full((16,), c, i32), jnp.full((16,), k, i32)])
                for j in range(width // 16):
                    s = pl.ds(j * 16, 16)
                    rows_v[k, s] = rows_v[k, s] * cf
                return _
            lax.fori_loop(0, CH, _edge, 0)
            pltpu.sync_copy(rows_v, acc_sh.at[dst_w.at[c]], add=True)
            return _
        lax.fori_loop(0, RPW, _chunk, 0)

        plsc.subcore_barrier()
        pltpu.sync_copy(acc_sh.at[pl.ds(sid * NROW_PT, NROW_PT)],
                        out.at[cid, pl.ds(sid * NROW_PT, NROW_PT)])

    return _pass


_sc_pass64 = _make_pass(H)
_sc_pass16 = _make_pass(16)

BR = 1000
_GRID = N // BR
_HIGH = lax.Precision.HIGHEST


def _dot(a, b):
    return jnp.dot(a, b, precision=_HIGH, preferred_element_type=f32)


def _tc_xw_body(x_ref, w_ref, o_ref):
    o_ref[...] = _dot(x_ref[...], w_ref[...])


def _tc_xw(x, w1):
    return pl.pallas_call(
        _tc_xw_body,
        grid=(_GRID,),
        in_specs=[
            pl.BlockSpec((BR, D_IN), lambda i: (i, 0)),
            pl.BlockSpec((D_IN, H), lambda i: (0, 0)),
        ],
        out_specs=pl.BlockSpec((BR, H), lambda i: (i, 0)),
        out_shape=jax.ShapeDtypeStruct((N, H), f32),
    )(x, w1)


def _tc_gcn1_body(p0, p1, xw, dis, b1, wg, h_o, m_o):
    d2 = dis[...] * dis[...]
    h = jnp.maximum(p0[...] + p1[...] + d2 * xw[...] + b1[...], 0.0)
    h_o[...] = h
    m_o[...] = _dot(h, wg[...])


def _tc_gcn1(p0, p1, xw, dis, b1, wg):
    rs = pl.BlockSpec((BR, H), lambda i: (i, 0))
    return pl.pallas_call(
        _tc_gcn1_body,
        grid=(_GRID,),
        in_specs=[
            rs, rs, rs,
            pl.BlockSpec((BR, 1), lambda i: (i, 0)),
            pl.BlockSpec((1, H), lambda i: (0, 0)),
            pl.BlockSpec((H, H), lambda i: (0, 0)),
        ],
        out_specs=[rs, rs],
        out_shape=[
            jax.ShapeDtypeStruct((N, H), f32),
            jax.ShapeDtypeStruct((N, H), f32),
        ],
    )(p0, p1, xw, dis, b1, wg)


def _tc_gru_body(p0, p1, h_ref, wihT, whhT, bih, bhh, wn, h_o, m_o):
    m = p0[...] + p1[...]
    h = h_ref[...]
    gi = _dot(m, wihT[...]) + bih[...]
    gh = _dot(h, whhT[...]) + bhh[...]
    ir, iz, inn = gi[:, :H], gi[:, H:2 * H], gi[:, 2 * H:]
    hr, hz, hn = gh[:, :H], gh[:, H:2 * H], gh[:, 2 * H:]
    r = jax.nn.sigmoid(ir + hr)
    z = jax.nn.sigmoid(iz + hz)
    ng = jnp.tanh(inn + r * hn)
    hnew = (1.0 - z) * ng + z * h
    h_o[...] = hnew
    m_o[...] = _dot(hnew, wn[...])


def _tc_gru(p0, p1, h, wihT, whhT, bih, bhh, wn):
    wout = wn.shape[1]
    rs = pl.BlockSpec((BR, H), lambda i: (i, 0))
    return pl.pallas_call(
        _tc_gru_body,
        grid=(_GRID,),
        in_specs=[
            rs, rs, rs,
            pl.BlockSpec((H, 3 * H), lambda i: (0, 0)),
            pl.BlockSpec((H, 3 * H), lambda i: (0, 0)),
            pl.BlockSpec((1, 3 * H), lambda i: (0, 0)),
            pl.BlockSpec((1, 3 * H), lambda i: (0, 0)),
            pl.BlockSpec((H, wout), lambda i: (0, 0)),
        ],
        out_specs=[rs, pl.BlockSpec((BR, wout), lambda i: (i, 0))],
        out_shape=[
            jax.ShapeDtypeStruct((N, H), f32),
            jax.ShapeDtypeStruct((N, wout), f32),
        ],
    )(p0, p1, h, wihT, whhT, bih, bhh, wn)


def _tc_final_body(p0, p1, hw, dis, b2, o_ref):
    d2 = dis[...] * dis[...]
    z = p0[...] + p1[...] + d2 * hw[...] + b2[...]
    col = lax.broadcasted_iota(i32, (BR, 16), 1)
    zm = jnp.where(col < C, z, -1e30)
    mx = jnp.max(zm, axis=1, keepdims=True)
    lse = jnp.log(jnp.sum(jnp.exp(zm - mx), axis=1, keepdims=True))
    o_ref[...] = z - (mx + lse)


def _tc_final(p0, p1, hw, dis, b2):
    rs = pl.BlockSpec((BR, 16), lambda i: (i, 0))
    return pl.pallas_call(
        _tc_final_body,
        grid=(_GRID,),
        in_specs=[
            rs, rs, rs,
            pl.BlockSpec((BR, 1), lambda i: (i, 0)),
            pl.BlockSpec((1, 16), lambda i: (0, 0)),
        ],
        out_specs=rs,
        out_shape=jax.ShapeDtypeStruct((N, 16), f32),
    )(p0, p1, hw, dis, b2)


def kernel(x, edge_index, edge_weight, W1, b1, Wg, Wih, bih, Whh, bhh, W2, b2):
    src = edge_index[0]
    dst = edge_index[1]
    pad = E_PAD - E
    src2d = jnp.concatenate([src, jnp.zeros((pad,), i32)]).reshape(ROWS, CH)
    dst2d = jnp.concatenate([dst, jnp.zeros((pad,), i32)]).reshape(ROWS, CH)
    ew2d = jnp.concatenate(
        [edge_weight, jnp.zeros((pad,), f32)]).reshape(ROWS, CH)

    dis_all, norm2d = _sc_prep(src2d, dst2d, ew2d)
    dis = dis_all[0, :N].reshape(N, 1)

    xw = _tc_xw(x, W1)
    agg1 = _sc_pass64(xw, src2d, dst2d, norm2d)[:, :N]
    h1, m1 = _tc_gcn1(agg1[0], agg1[1], xw, dis, b1.reshape(1, H), Wg[0])

    wihT = Wih.T
    whhT = Whh.T
    bih2 = bih.reshape(1, 3 * H)
    bhh2 = bhh.reshape(1, 3 * H)

    aggm1 = _sc_pass64(m1, src2d, dst2d, ew2d)[:, :N]
    h2, m2 = _tc_gru(aggm1[0], aggm1[1], h1, wihT, whhT, bih2, bhh2, Wg[1])

    aggm2 = _sc_pass64(m2, src2d, dst2d, ew2d)[:, :N]
    w2pad = jnp.pad(W2, ((0, 0), (0, 16 - C)))
    _, hw2 = _tc_gru(aggm2[0], aggm2[1], h2, wihT, whhT, bih2, bhh2, w2pad)

    agg2 = _sc_pass16(hw2, src2d, dst2d, norm2d)[:, :N]
    b2pad = jnp.pad(b2, (0, 16 - C)).reshape(1, 16)
    out = _tc_final(agg2[0], agg2[1], hw2, dis, b2pad)
    return out[:, :C]
